# Initial kernel scaffold; baseline (speedup 1.0000x reference)
import jax
import jax.numpy as jnp
from jax.experimental import pallas as pl


def kernel(x, edge_index, batch, Wl, Wr):
    raise NotImplementedError("write your pallas kernel here")


# speedup vs baseline: 6.4324x; 6.4324x over previous
import jax
import jax.numpy as jnp
from jax import lax
from jax.experimental import pallas as pl
from jax.experimental.pallas import tpu as pltpu
from jax.experimental.pallas import tpu_sc as plsc

_N = 10000
_E = 320000
_FIN = 128
_FOUT = 64
_G = 64
_NC = 2
_NS = 16
_NW = _NC * _NS
_EW = _E // _NW
_CH = 80
_NCHUNK = _EW // _CH
_GC = 25
_NG = _NCHUNK // _GC
_NP = 10240
_RT = _NP // _NS
_BLK = 1000
_NBLK = _N // _BLK


def _sc_body(x_hbm, src_hbm, dst_hbm, hpart_hbm, cpart_hbm,
             src_v, dst_v, rows_v, ones_v, zrow_v, zcnt_v, h_sh, c_sh, sem):
    c = lax.axis_index("c")
    s = lax.axis_index("s")
    base = s * _RT

    def _fill_ones(i, _):
        ones_v[i, pl.ds(0, 16)] = jnp.ones((16,), jnp.float32)
        return 0
    lax.fori_loop(0, _CH, _fill_ones, 0)

    def _zrow(i, _):
        for j in range(_FIN // 16):
            zrow_v[i, pl.ds(j * 16, 16)] = jnp.zeros((16,), jnp.float32)
        return 0
    lax.fori_loop(0, 64, _zrow, 0)

    def _zcnt(i, _):
        zcnt_v[i, pl.ds(0, 16)] = jnp.zeros((16,), jnp.float32)
        return 0
    lax.fori_loop(0, 64, _zcnt, 0)

    for k in range(_RT // 64):
        pltpu.sync_copy(zrow_v, h_sh.at[pl.ds(base + k * 64, 64)])
        pltpu.sync_copy(zcnt_v, c_sh.at[pl.ds(base + k * 64, 64)])
    plsc.subcore_barrier()

    def _group(g, _):
        pltpu.sync_copy(src_hbm.at[c, s, g], src_v)
        pltpu.sync_copy(dst_hbm.at[c, s, g], dst_v)

        def _step(i, _):
            pltpu.async_copy(x_hbm.at[src_v.at[i]], rows_v, sem).wait()
            pltpu.sync_copy(rows_v, h_sh.at[dst_v.at[i]], add=True)
            pltpu.sync_copy(ones_v, c_sh.at[dst_v.at[i]], add=True)
            return 0
        lax.fori_loop(0, _GC, _step, 0)
        return 0
    lax.fori_loop(0, _NG, _group, 0)
    plsc.subcore_barrier()

    for k in range(_RT // 64):
        pltpu.sync_copy(h_sh.at[pl.ds(base + k * 64, 64)], zrow_v)
        pltpu.sync_copy(zrow_v, hpart_hbm.at[c, pl.ds(base + k * 64, 64)])
        pltpu.sync_copy(c_sh.at[pl.ds(base + k * 64, 64)], zcnt_v)
        pltpu.sync_copy(zcnt_v, cpart_hbm.at[c, pl.ds(base + k * 64, 64)])


_sc_scatter = pl.kernel(
    _sc_body,
    out_type=(jax.ShapeDtypeStruct((_NC, _NP, _FIN), jnp.float32),
              jax.ShapeDtypeStruct((_NC, _NP, 16), jnp.float32)),
    mesh=plsc.VectorSubcoreMesh(core_axis_name="c", subcore_axis_name="s"),
    scratch_types=[
        pltpu.VMEM((_GC, _CH), jnp.int32),
        pltpu.VMEM((_GC, _CH), jnp.int32),
        pltpu.VMEM((_CH, _FIN), jnp.float32),
        pltpu.VMEM((_CH, 16), jnp.float32),
        pltpu.VMEM((64, _FIN), jnp.float32),
        pltpu.VMEM((64, 16), jnp.float32),
        pltpu.VMEM_SHARED((_NP, _FIN), jnp.float32),
        pltpu.VMEM_SHARED((_NP, 16), jnp.float32),
        pltpu.SemaphoreType.DMA,
    ],
    compiler_params=pltpu.CompilerParams(use_tc_tiling_on_sc=False),
)


def _tc_body(x_ref, h_ref, c_ref, b_ref, wl_ref, wr_ref, out_ref):
    i = pl.program_id(0)
    xb = x_ref[...]
    hb = h_ref[0] + h_ref[1]
    cnt = c_ref[0, :, 0:1] + c_ref[1, :, 0:1]
    deg = jnp.minimum(cnt, 4.0)
    acc = jnp.zeros((_BLK, _FOUT), jnp.float32)
    for d in range(5):
        m = (deg == float(d)).astype(jnp.float32)
        acc = acc + jnp.dot(hb * m, wl_ref[d], preferred_element_type=jnp.float32)
        acc = acc + jnp.dot(xb * m, wr_ref[d], preferred_element_type=jnp.float32)
    acc = jnp.maximum(acc, 0.0)
    acc = acc - jnp.max(acc, axis=1, keepdims=True)
    e = jnp.exp(acc)
    p = e / jnp.sum(e, axis=1, keepdims=True)
    bb = b_ref[0, 0, :]
    oh = (bb[:, None] == lax.broadcasted_iota(jnp.int32, (_BLK, _G), 1))
    contrib = lax.dot_general(oh.astype(jnp.float32), p,
                              (((0,), (0,)), ((), ())),
                              preferred_element_type=jnp.float32)

    @pl.when(i == 0)
    def _():
        out_ref[...] = jnp.zeros_like(out_ref)

    out_ref[...] += contrib


def kernel(x, edge_index, batch, Wl, Wr):
    src = edge_index[0].reshape(_NC, _NS, _NG, _GC, _CH)
    dst = edge_index[1].reshape(_NC, _NS, _NG, _GC, _CH)
    hpart, cpart = _sc_scatter(x, src, dst)
    batch3 = batch.reshape(_NBLK, 1, _BLK)
    pooled = pl.pallas_call(
        _tc_body,
        grid=(_NBLK,),
        in_specs=[
            pl.BlockSpec((_BLK, _FIN), lambda b: (b, 0)),
            pl.BlockSpec((_NC, _BLK, _FIN), lambda b: (0, b, 0)),
            pl.BlockSpec((_NC, _BLK, 16), lambda b: (0, b, 0)),
            pl.BlockSpec((1, 1, _BLK), lambda b: (b, 0, 0)),
            pl.BlockSpec((5, _FIN, _FOUT), lambda b: (0, 0, 0)),
            pl.BlockSpec((5, _FIN, _FOUT), lambda b: (0, 0, 0)),
        ],
        out_specs=pl.BlockSpec((_G, _FOUT), lambda b: (0, 0)),
        out_shape=jax.ShapeDtypeStruct((_G, _FOUT), jnp.float32),
    )(x, hpart, cpart, batch3, Wl, Wr)
    return pooled


# speedup vs baseline: 9.3469x; 1.4531x over previous
import jax
import jax.numpy as jnp
from jax import lax
from jax.experimental import pallas as pl
from jax.experimental.pallas import tpu as pltpu
from jax.experimental.pallas import tpu_sc as plsc

_N = 10000
_E = 320000
_FIN = 128
_FOUT = 64
_G = 64
_NC = 2
_NS = 16
_NW = _NC * _NS
_EW = _E // _NW
_CH = 100
_NCHUNK = _EW // _CH
_GC = 20
_NG = _NCHUNK // _GC
_NP = 10240
_RT = _NP // _NS
_BLK = 1000
_NBLK = _N // _BLK


def _sc_body(x_hbm, src_hbm, dst_hbm, hpart_hbm, cpart_hbm,
             src_v, dst_v, rows0_v, rows1_v, ones_v, zrow_v, zcnt_v,
             h_sh, c_sh, sem0, sem1):
    c = lax.axis_index("c")
    s = lax.axis_index("s")
    base = s * _RT

    def _fill_ones(i, _):
        ones_v[i] = jnp.ones((16,), jnp.float32)
        return 0
    lax.fori_loop(0, _CH, _fill_ones, 0)

    def _zrow(i, _):
        for j in range(_FIN // 16):
            zrow_v[i, pl.ds(j * 16, 16)] = jnp.zeros((16,), jnp.float32)
        return 0
    lax.fori_loop(0, 32, _zrow, 0)

    def _zcnt(i, _):
        zcnt_v[i] = jnp.zeros((16,), jnp.float32)
        return 0
    lax.fori_loop(0, 64, _zcnt, 0)

    for k in range(_RT // 32):
        pltpu.sync_copy(zrow_v, h_sh.at[pl.ds(base + k * 32, 32)])
    for k in range(_RT // 64):
        pltpu.sync_copy(zcnt_v, c_sh.at[pl.ds(base + k * 64, 64)])
    plsc.subcore_barrier()

    def _group(g, _):
        pltpu.sync_copy(src_hbm.at[c, s, g], src_v)
        pltpu.sync_copy(dst_hbm.at[c, s, g], dst_v)

        pltpu.async_copy(x_hbm.at[src_v.at[0]], rows0_v, sem0)

        def _pair(j, _):
            i0 = 2 * j
            pltpu.async_copy(x_hbm.at[src_v.at[i0 + 1]], rows1_v, sem1)
            pltpu.make_async_copy(x_hbm.at[pl.ds(0, _CH)], rows0_v, sem0).wait()
            pltpu.sync_copy(rows0_v, h_sh.at[dst_v.at[i0]], add=True)
            pltpu.sync_copy(ones_v, c_sh.at[dst_v.at[i0]], add=True)

            @pl.when(j < _GC // 2 - 1)
            def _():
                pltpu.async_copy(x_hbm.at[src_v.at[i0 + 2]], rows0_v, sem0)

            pltpu.make_async_copy(x_hbm.at[pl.ds(0, _CH)], rows1_v, sem1).wait()
            pltpu.sync_copy(rows1_v, h_sh.at[dst_v.at[i0 + 1]], add=True)
            pltpu.sync_copy(ones_v, c_sh.at[dst_v.at[i0 + 1]], add=True)
            return 0
        lax.fori_loop(0, _GC // 2, _pair, 0)
        return 0
    lax.fori_loop(0, _NG, _group, 0)
    plsc.subcore_barrier()

    for k in range(_RT // 32):
        pltpu.sync_copy(h_sh.at[pl.ds(base + k * 32, 32)], zrow_v)
        pltpu.sync_copy(zrow_v, hpart_hbm.at[c, pl.ds(base + k * 32, 32)])
    for k in range(_RT // 64):
        pltpu.sync_copy(c_sh.at[pl.ds(base + k * 64, 64)], zcnt_v)
        pltpu.sync_copy(zcnt_v, cpart_hbm.at[c, pl.ds(base + k * 64, 64)])


_sc_scatter = pl.kernel(
    _sc_body,
    out_type=(jax.ShapeDtypeStruct((_NC, _NP, _FIN), jnp.float32),
              jax.ShapeDtypeStruct((_NC, _NP, 16), jnp.float32)),
    mesh=plsc.VectorSubcoreMesh(core_axis_name="c", subcore_axis_name="s"),
    scratch_types=[
        pltpu.VMEM((_GC, _CH), jnp.int32),
        pltpu.VMEM((_GC, _CH), jnp.int32),
        pltpu.VMEM((_CH, _FIN), jnp.float32),
        pltpu.VMEM((_CH, _FIN), jnp.float32),
        pltpu.VMEM((_CH, 16), jnp.float32),
        pltpu.VMEM((32, _FIN), jnp.float32),
        pltpu.VMEM((64, 16), jnp.float32),
        pltpu.VMEM_SHARED((_NP, _FIN), jnp.float32),
        pltpu.VMEM_SHARED((_NP, 16), jnp.float32),
        pltpu.SemaphoreType.DMA,
        pltpu.SemaphoreType.DMA,
    ],
    compiler_params=pltpu.CompilerParams(use_tc_tiling_on_sc=False),
)


def _tc_body(x_ref, h_ref, c_ref, b_ref, wl_ref, wr_ref, out_ref):
    i = pl.program_id(0)
    xb = x_ref[...]
    hb = h_ref[0] + h_ref[1]
    cnt = c_ref[0, :, 0:1] + c_ref[1, :, 0:1]
    deg = jnp.minimum(cnt, 4.0)
    acc = jnp.zeros((_BLK, _FOUT), jnp.float32)
    for d in range(5):
        m = (deg == float(d)).astype(jnp.float32)
        acc = acc + jnp.dot(hb * m, wl_ref[d], preferred_element_type=jnp.float32)
        acc = acc + jnp.dot(xb * m, wr_ref[d], preferred_element_type=jnp.float32)
    acc = jnp.maximum(acc, 0.0)
    acc = acc - jnp.max(acc, axis=1, keepdims=True)
    e = jnp.exp(acc)
    p = e / jnp.sum(e, axis=1, keepdims=True)
    bb = b_ref[0, 0, :]
    oh = (bb[:, None] == lax.broadcasted_iota(jnp.int32, (_BLK, _G), 1))
    contrib = lax.dot_general(oh.astype(jnp.float32), p,
                              (((0,), (0,)), ((), ())),
                              preferred_element_type=jnp.float32)

    @pl.when(i == 0)
    def _():
        out_ref[...] = jnp.zeros_like(out_ref)

    out_ref[...] += contrib


def kernel(x, edge_index, batch, Wl, Wr):
    src = edge_index[0].reshape(_NC, _NS, _NG, _GC, _CH)
    dst = edge_index[1].reshape(_NC, _NS, _NG, _GC, _CH)
    hpart, cpart = _sc_scatter(x, src, dst)
    batch3 = batch.reshape(_NBLK, 1, _BLK)
    pooled = pl.pallas_call(
        _tc_body,
        grid=(_NBLK,),
        in_specs=[
            pl.BlockSpec((_BLK, _FIN), lambda b: (b, 0)),
            pl.BlockSpec((_NC, _BLK, _FIN), lambda b: (0, b, 0)),
            pl.BlockSpec((_NC, _BLK, 16), lambda b: (0, b, 0)),
            pl.BlockSpec((1, 1, _BLK), lambda b: (b, 0, 0)),
            pl.BlockSpec((5, _FIN, _FOUT), lambda b: (0, 0, 0)),
            pl.BlockSpec((5, _FIN, _FOUT), lambda b: (0, 0, 0)),
        ],
        out_specs=pl.BlockSpec((_G, _FOUT), lambda b: (0, 0)),
        out_shape=jax.ShapeDtypeStruct((_G, _FOUT), jnp.float32),
    )(x, hpart, cpart, batch3, Wl, Wr)
    return pooled
